# baseline (speedup 1.0000x reference)
import math

import jax
import jax.numpy as jnp
from jax.experimental import pallas as pl

NUM_TOKENS = 32000
DIM = 1024
DEPTH = 2
SEQ = 2048
HEADS = 16
DH = DIM // HEADS
BATCH = 2
BUCKET = 64
NB = SEQ // BUCKET
NC = SEQ // BUCKET

bf16 = jnp.bfloat16
f32 = jnp.float32


def _layernorm(x, g, b, eps=1e-5):
    mu = jnp.mean(x, axis=-1, keepdims=True)
    var = jnp.var(x, axis=-1, keepdims=True)
    return (x - mu) / jnp.sqrt(var + eps) * g + b


def _lsh_attention_ref(qk, v, rot):
    B, H, S, Dh = qk.shape
    rotated = jnp.einsum('bhsd,dr->bhsr', qk, rot)
    rotated = jnp.concatenate([rotated, -rotated], axis=-1)
    buckets = jnp.argmax(rotated, axis=-1)
    ticker = jnp.arange(S)
    sort_key = buckets * S + ticker[None, None, :]
    sidx = jnp.argsort(sort_key, axis=-1)
    spos = sidx
    sqk = jnp.take_along_axis(qk, sidx[..., None], axis=2)
    sv = jnp.take_along_axis(v, sidx[..., None], axis=2)
    C = S // BUCKET
    bq = sqk.reshape(B, H, C, BUCKET, Dh)
    bk = sqk / (jnp.linalg.norm(sqk, axis=-1, keepdims=True) + 1e-9)
    bk = bk.reshape(B, H, C, BUCKET, Dh)
    bv = sv.reshape(B, H, C, BUCKET, Dh)
    bpos = spos.reshape(B, H, C, BUCKET)

    def look_back(t):
        return jnp.concatenate([t, jnp.roll(t, 1, axis=2)], axis=3)

    bk2 = look_back(bk)
    bv2 = look_back(bv)
    kpos = look_back(bpos)
    dots = jnp.einsum('bhcid,bhcjd->bhcij', bq, bk2) / math.sqrt(Dh)
    qpos = bpos
    causal_mask = kpos[:, :, :, None, :] > qpos[:, :, :, :, None]
    dots = jnp.where(causal_mask, -1e9, dots)
    self_mask = kpos[:, :, :, None, :] == qpos[:, :, :, :, None]
    dots = jnp.where(self_mask, -1e5, dots)
    attn = jax.nn.softmax(dots, axis=-1)
    o = jnp.einsum('bhcij,bhcjd->bhcid', attn, bv2).reshape(B, H, S, Dh)
    tick = jnp.broadcast_to(ticker[None, None, :], sidx.shape)
    undo = jnp.put_along_axis(jnp.zeros_like(sidx), sidx, tick, axis=-1,
                              inplace=False)
    return jnp.take_along_axis(o, undo[..., None], axis=2)


def _attn_kernel(sqk_ref, sv_ref, posr_ref, posc_ref, out_ref):
    sqk = sqk_ref[0]
    sv = sv_ref[0]
    posr = posr_ref[0]
    posc = posc_ref[0]
    nrm = jnp.sqrt(jnp.sum(sqk * sqk, axis=-1, keepdims=True)) + 1e-9
    bkn = sqk / nrm
    ocs = []
    for c in range(NC):
        p = NC - 1 if c == 0 else c - 1
        sl = slice(c * BUCKET, (c + 1) * BUCKET)
        slp = slice(p * BUCKET, (p + 1) * BUCKET)
        bq = sqk[sl]
        bk2 = jnp.concatenate([bkn[sl], bkn[slp]], axis=0)
        bv2 = jnp.concatenate([sv[sl], sv[slp]], axis=0)
        kpos = jnp.concatenate([posr[:, sl], posr[:, slp]], axis=1)
        qpos = posc[sl]
        dots = jax.lax.dot_general(
            bq.astype(bf16), bk2.astype(bf16),
            (((1,), (1,)), ((), ())), preferred_element_type=f32) / 8.0
        dots = jnp.where(kpos > qpos, -1e9, dots)
        dots = jnp.where(kpos == qpos, -1e5, dots)
        attn = jax.nn.softmax(dots, axis=-1)
        ocs.append(jax.lax.dot_general(
            attn.astype(bf16), bv2.astype(bf16),
            (((1,), (0,)), ((), ())), preferred_element_type=f32))
    o_s = jnp.concatenate(ocs, axis=0)
    row_iota = jax.lax.broadcasted_iota(jnp.int32, (SEQ, SEQ), 0)
    U = (posr.astype(jnp.int32) == row_iota).astype(bf16)
    hi = o_s.astype(bf16)
    r1 = o_s - hi.astype(f32)
    mid = r1.astype(bf16)
    lo = (r1 - mid.astype(f32)).astype(bf16)
    o_un = jax.lax.dot_general(U, hi, (((1,), (0,)), ((), ())),
                               preferred_element_type=f32)
    o_un = o_un + jax.lax.dot_general(U, mid, (((1,), (0,)), ((), ())),
                                      preferred_element_type=f32)
    o_un = o_un + jax.lax.dot_general(U, lo, (((1,), (0,)), ((), ())),
                                      preferred_element_type=f32)
    out_ref[0] = o_un


def _attn1(sqk, sv, posf):
    BH = sqk.shape[0]
    posr = posf.reshape(BH, 1, SEQ)
    posc = posf.reshape(BH, SEQ, 1)
    return pl.pallas_call(
        _attn_kernel,
        grid=(BH,),
        in_specs=[
            pl.BlockSpec((1, SEQ, DH), lambda i: (i, 0, 0)),
            pl.BlockSpec((1, SEQ, DH), lambda i: (i, 0, 0)),
            pl.BlockSpec((1, 1, SEQ), lambda i: (i, 0, 0)),
            pl.BlockSpec((1, SEQ, 1), lambda i: (i, 0, 0)),
        ],
        out_specs=pl.BlockSpec((1, SEQ, DH), lambda i: (i, 0, 0)),
        out_shape=jax.ShapeDtypeStruct((BH, SEQ, DH), f32),
    )(sqk, sv, posr, posc)


def _tail_kernel(x_ref, o_ref, wo_ref, l2g_ref, l2b_ref, w1_ref, b1_ref,
                 w2_ref, b2_ref, lfg_ref, lfb_ref, wd_ref, bd_ref,
                 wout_ref, bout_ref, out_ref):
    x = x_ref[...]
    o = o_ref[...]
    x = x + jnp.dot(o.astype(bf16), wo_ref[...], preferred_element_type=f32)
    h = _layernorm(x, l2g_ref[...], l2b_ref[...])
    p1 = jnp.dot(h.astype(bf16), w1_ref[...], preferred_element_type=f32) + b1_ref[...]
    g = jax.nn.gelu(p1)
    x = x + (jnp.dot(g.astype(bf16), w2_ref[...], preferred_element_type=f32) + b2_ref[...])
    h = _layernorm(x, lfg_ref[...], lfb_ref[...])
    hh = jax.nn.gelu(jnp.dot(h.astype(bf16), wd_ref[...],
                             preferred_element_type=f32) + bd_ref[...])
    out_ref[...] = jnp.dot(hh.astype(bf16), wout_ref[...],
                           preferred_element_type=f32) + bout_ref[...]


def _tail(x, o, lp, params):
    T = x.shape[0]
    wout_p = jnp.zeros((4 * DIM, 128), f32).at[:, :2].set(params['Wout']).astype(bf16)
    bout_p = jnp.zeros((128,), f32).at[:2].set(params['bout'])
    blk = 256
    return pl.pallas_call(
        _tail_kernel,
        grid=(T // blk,),
        in_specs=[
            pl.BlockSpec((blk, DIM), lambda i: (i, 0)),
            pl.BlockSpec((blk, DIM), lambda i: (i, 0)),
            pl.BlockSpec((DIM, DIM), lambda i: (0, 0)),
            pl.BlockSpec((DIM,), lambda i: (0,)),
            pl.BlockSpec((DIM,), lambda i: (0,)),
            pl.BlockSpec((DIM, 4 * DIM), lambda i: (0, 0)),
            pl.BlockSpec((4 * DIM,), lambda i: (0,)),
            pl.BlockSpec((4 * DIM, DIM), lambda i: (0, 0)),
            pl.BlockSpec((DIM,), lambda i: (0,)),
            pl.BlockSpec((DIM,), lambda i: (0,)),
            pl.BlockSpec((DIM,), lambda i: (0,)),
            pl.BlockSpec((DIM, 4 * DIM), lambda i: (0, 0)),
            pl.BlockSpec((4 * DIM,), lambda i: (0,)),
            pl.BlockSpec((4 * DIM, 128), lambda i: (0, 0)),
            pl.BlockSpec((128,), lambda i: (0,)),
        ],
        out_specs=pl.BlockSpec((blk, 128), lambda i: (i, 0)),
        out_shape=jax.ShapeDtypeStruct((T, 128), f32),
    )(x, o, lp['Wo'].astype(bf16), lp['ln2_g'], lp['ln2_b'],
      lp['W1'].astype(bf16), lp['b1'], lp['W2'].astype(bf16), lp['b2'],
      params['lnf_g'], params['lnf_b'], params['Wd'].astype(bf16),
      params['bd'], wout_p, bout_p)


def kernel(input_ids, params):
    x = params['tok_emb'][input_ids] + params['pos_emb'][None, :, :]
    B, S, D = x.shape

    lp = params['layers'][0]
    h = _layernorm(x, lp['ln1_g'], lp['ln1_b'])
    qk = (h @ lp['Wqk']).reshape(B, S, HEADS, DH).transpose(0, 2, 1, 3)
    v = (h @ lp['Wv']).reshape(B, S, HEADS, DH).transpose(0, 2, 1, 3)
    o = _lsh_attention_ref(qk, v, lp['rot'])
    o = o.transpose(0, 2, 1, 3).reshape(B, S, HEADS * DH) @ lp['Wo']
    x = x + o
    h = _layernorm(x, lp['ln2_g'], lp['ln2_b'])
    x = x + (jax.nn.gelu(h @ lp['W1'] + lp['b1']) @ lp['W2'] + lp['b2'])

    lp = params['layers'][1]
    h = _layernorm(x, lp['ln1_g'], lp['ln1_b'])
    qk = (h @ lp['Wqk']).reshape(B, S, HEADS, DH).transpose(0, 2, 1, 3)
    v = (h @ lp['Wv']).reshape(B, S, HEADS, DH).transpose(0, 2, 1, 3)
    rotated = jnp.einsum('bhsd,dr->bhsr', qk, lp['rot'])
    rotated = jnp.concatenate([rotated, -rotated], axis=-1)
    buckets = jnp.argmax(rotated, axis=-1)
    ticker = jnp.arange(S)
    sort_key = buckets * S + ticker[None, None, :]
    sidx = jnp.argsort(sort_key, axis=-1)
    sqk = jnp.take_along_axis(qk, sidx[..., None], axis=2)
    sv = jnp.take_along_axis(v, sidx[..., None], axis=2)

    sqk_b, sv_b, pos_b, x_b = jax.lax.optimization_barrier(
        (sqk, sv, sidx.astype(f32), x))
    o1 = _attn1(sqk_b.reshape(B * HEADS, S, DH), sv_b.reshape(B * HEADS, S, DH),
                pos_b.reshape(B * HEADS, S))
    o1 = o1.reshape(B, HEADS, S, DH).transpose(0, 2, 1, 3).reshape(B * S, DIM)
    logits = _tail(x_b.reshape(B * S, D), o1, lp, params).reshape(B, S, 128)
    return (logits[..., 0], logits[..., 1])


# speedup vs baseline: 1.0521x; 1.0521x over previous
import math

import jax
import jax.numpy as jnp
from jax.experimental import pallas as pl

NUM_TOKENS = 32000
DIM = 1024
DEPTH = 2
SEQ = 2048
HEADS = 16
DH = DIM // HEADS
BATCH = 2
BUCKET = 64
NB = SEQ // BUCKET
NC = SEQ // BUCKET

bf16 = jnp.bfloat16
f32 = jnp.float32


def _layernorm(x, g, b, eps=1e-5):
    mu = jnp.mean(x, axis=-1, keepdims=True)
    var = jnp.var(x, axis=-1, keepdims=True)
    return (x - mu) / jnp.sqrt(var + eps) * g + b


def _lsh_attention_ref(qk, v, rot):
    B, H, S, Dh = qk.shape
    rotated = jnp.einsum('bhsd,dr->bhsr', qk, rot)
    rotated = jnp.concatenate([rotated, -rotated], axis=-1)
    buckets = jnp.argmax(rotated, axis=-1)
    ticker = jnp.arange(S)
    sort_key = buckets * S + ticker[None, None, :]
    sidx = jnp.argsort(sort_key, axis=-1)
    spos = sidx
    sqk = jnp.take_along_axis(qk, sidx[..., None], axis=2)
    sv = jnp.take_along_axis(v, sidx[..., None], axis=2)
    C = S // BUCKET
    bq = sqk.reshape(B, H, C, BUCKET, Dh)
    bk = sqk / (jnp.linalg.norm(sqk, axis=-1, keepdims=True) + 1e-9)
    bk = bk.reshape(B, H, C, BUCKET, Dh)
    bv = sv.reshape(B, H, C, BUCKET, Dh)
    bpos = spos.reshape(B, H, C, BUCKET)

    def look_back(t):
        return jnp.concatenate([t, jnp.roll(t, 1, axis=2)], axis=3)

    bk2 = look_back(bk)
    bv2 = look_back(bv)
    kpos = look_back(bpos)
    dots = jnp.einsum('bhcid,bhcjd->bhcij', bq, bk2) / math.sqrt(Dh)
    qpos = bpos
    causal_mask = kpos[:, :, :, None, :] > qpos[:, :, :, :, None]
    dots = jnp.where(causal_mask, -1e9, dots)
    self_mask = kpos[:, :, :, None, :] == qpos[:, :, :, :, None]
    dots = jnp.where(self_mask, -1e5, dots)
    attn = jax.nn.softmax(dots, axis=-1)
    o = jnp.einsum('bhcij,bhcjd->bhcid', attn, bv2).reshape(B, H, S, Dh)
    undo = jnp.argsort(sidx, axis=-1)
    return jnp.take_along_axis(o, undo[..., None], axis=2)


def _attn_kernel(sqk_ref, sv_ref, posr_ref, posc_ref, out_ref):
    sqk = sqk_ref[0]
    sv = sv_ref[0]
    posr = posr_ref[0]
    posc = posc_ref[0]
    nrm = jnp.sqrt(jnp.sum(sqk * sqk, axis=-1, keepdims=True)) + 1e-9
    bkn = sqk / nrm
    ocs = []
    for c in range(NC):
        p = NC - 1 if c == 0 else c - 1
        sl = slice(c * BUCKET, (c + 1) * BUCKET)
        slp = slice(p * BUCKET, (p + 1) * BUCKET)
        bq = sqk[sl]
        bk2 = jnp.concatenate([bkn[sl], bkn[slp]], axis=0)
        bv2 = jnp.concatenate([sv[sl], sv[slp]], axis=0)
        kpos = jnp.concatenate([posr[:, sl], posr[:, slp]], axis=1)
        qpos = posc[sl]
        dots = jax.lax.dot_general(
            bq.astype(bf16), bk2.astype(bf16),
            (((1,), (1,)), ((), ())), preferred_element_type=f32) / 8.0
        dots = jnp.where(kpos > qpos, -1e9, dots)
        dots = jnp.where(kpos == qpos, -1e5, dots)
        attn = jax.nn.softmax(dots, axis=-1)
        ocs.append(jax.lax.dot_general(
            attn.astype(bf16), bv2.astype(bf16),
            (((1,), (0,)), ((), ())), preferred_element_type=f32))
    o_s = jnp.concatenate(ocs, axis=0)
    row_iota = jax.lax.broadcasted_iota(jnp.int32, (SEQ, SEQ), 0)
    U = (posr.astype(jnp.int32) == row_iota).astype(bf16)
    hi = o_s.astype(bf16)
    r1 = o_s - hi.astype(f32)
    mid = r1.astype(bf16)
    lo = (r1 - mid.astype(f32)).astype(bf16)
    o_un = jax.lax.dot_general(U, hi, (((1,), (0,)), ((), ())),
                               preferred_element_type=f32)
    o_un = o_un + jax.lax.dot_general(U, mid, (((1,), (0,)), ((), ())),
                                      preferred_element_type=f32)
    o_un = o_un + jax.lax.dot_general(U, lo, (((1,), (0,)), ((), ())),
                                      preferred_element_type=f32)
    out_ref[0] = o_un


def _attn1(sqk, sv, posf):
    BH = sqk.shape[0]
    posr = posf.reshape(BH, 1, SEQ)
    posc = posf.reshape(BH, SEQ, 1)
    return pl.pallas_call(
        _attn_kernel,
        grid=(BH,),
        in_specs=[
            pl.BlockSpec((1, SEQ, DH), lambda i: (i, 0, 0)),
            pl.BlockSpec((1, SEQ, DH), lambda i: (i, 0, 0)),
            pl.BlockSpec((1, 1, SEQ), lambda i: (i, 0, 0)),
            pl.BlockSpec((1, SEQ, 1), lambda i: (i, 0, 0)),
        ],
        out_specs=pl.BlockSpec((1, SEQ, DH), lambda i: (i, 0, 0)),
        out_shape=jax.ShapeDtypeStruct((BH, SEQ, DH), f32),
    )(sqk, sv, posr, posc)


def _tail_kernel(x_ref, o_ref, wo_ref, l2g_ref, l2b_ref, w1_ref, b1_ref,
                 w2_ref, b2_ref, lfg_ref, lfb_ref, wd_ref, bd_ref,
                 wout_ref, bout_ref, out_ref):
    x = x_ref[...]
    o = o_ref[...]
    x = x + jnp.dot(o.astype(bf16), wo_ref[...], preferred_element_type=f32)
    h = _layernorm(x, l2g_ref[...], l2b_ref[...])
    p1 = jnp.dot(h.astype(bf16), w1_ref[...], preferred_element_type=f32) + b1_ref[...]
    g = jax.nn.gelu(p1)
    x = x + (jnp.dot(g.astype(bf16), w2_ref[...], preferred_element_type=f32) + b2_ref[...])
    h = _layernorm(x, lfg_ref[...], lfb_ref[...])
    hh = jax.nn.gelu(jnp.dot(h.astype(bf16), wd_ref[...],
                             preferred_element_type=f32) + bd_ref[...])
    out_ref[...] = jnp.dot(hh.astype(bf16), wout_ref[...],
                           preferred_element_type=f32) + bout_ref[...]


def _tail(x, o, lp, params):
    T = x.shape[0]
    wout_p = jnp.zeros((4 * DIM, 128), f32).at[:, :2].set(params['Wout']).astype(bf16)
    bout_p = jnp.zeros((128,), f32).at[:2].set(params['bout'])
    blk = 256
    return pl.pallas_call(
        _tail_kernel,
        grid=(T // blk,),
        in_specs=[
            pl.BlockSpec((blk, DIM), lambda i: (i, 0)),
            pl.BlockSpec((blk, DIM), lambda i: (i, 0)),
            pl.BlockSpec((DIM, DIM), lambda i: (0, 0)),
            pl.BlockSpec((DIM,), lambda i: (0,)),
            pl.BlockSpec((DIM,), lambda i: (0,)),
            pl.BlockSpec((DIM, 4 * DIM), lambda i: (0, 0)),
            pl.BlockSpec((4 * DIM,), lambda i: (0,)),
            pl.BlockSpec((4 * DIM, DIM), lambda i: (0, 0)),
            pl.BlockSpec((DIM,), lambda i: (0,)),
            pl.BlockSpec((DIM,), lambda i: (0,)),
            pl.BlockSpec((DIM,), lambda i: (0,)),
            pl.BlockSpec((DIM, 4 * DIM), lambda i: (0, 0)),
            pl.BlockSpec((4 * DIM,), lambda i: (0,)),
            pl.BlockSpec((4 * DIM, 128), lambda i: (0, 0)),
            pl.BlockSpec((128,), lambda i: (0,)),
        ],
        out_specs=pl.BlockSpec((blk, 128), lambda i: (i, 0)),
        out_shape=jax.ShapeDtypeStruct((T, 128), f32),
    )(x, o, lp['Wo'].astype(bf16), lp['ln2_g'], lp['ln2_b'],
      lp['W1'].astype(bf16), lp['b1'], lp['W2'].astype(bf16), lp['b2'],
      params['lnf_g'], params['lnf_b'], params['Wd'].astype(bf16),
      params['bd'], wout_p, bout_p)


def kernel(input_ids, params):
    x = params['tok_emb'][input_ids] + params['pos_emb'][None, :, :]
    B, S, D = x.shape

    lp = params['layers'][0]
    h = _layernorm(x, lp['ln1_g'], lp['ln1_b'])
    qk = (h @ lp['Wqk']).reshape(B, S, HEADS, DH).transpose(0, 2, 1, 3)
    v = (h @ lp['Wv']).reshape(B, S, HEADS, DH).transpose(0, 2, 1, 3)
    o = _lsh_attention_ref(qk, v, lp['rot'])
    o = o.transpose(0, 2, 1, 3).reshape(B, S, HEADS * DH) @ lp['Wo']
    x = x + o
    h = _layernorm(x, lp['ln2_g'], lp['ln2_b'])
    x = x + (jax.nn.gelu(h @ lp['W1'] + lp['b1']) @ lp['W2'] + lp['b2'])

    lp = params['layers'][1]
    h = _layernorm(x, lp['ln1_g'], lp['ln1_b'])
    qk = (h @ lp['Wqk']).reshape(B, S, HEADS, DH).transpose(0, 2, 1, 3)
    v = (h @ lp['Wv']).reshape(B, S, HEADS, DH).transpose(0, 2, 1, 3)
    rotated = jnp.einsum('bhsd,dr->bhsr', qk, lp['rot'])
    rotated = jnp.concatenate([rotated, -rotated], axis=-1)
    buckets = jnp.argmax(rotated, axis=-1)
    ticker = jnp.arange(S)
    sort_key = buckets * S + ticker[None, None, :]
    sidx = jnp.argsort(sort_key, axis=-1)
    sqk = jnp.take_along_axis(qk, sidx[..., None], axis=2)
    sv = jnp.take_along_axis(v, sidx[..., None], axis=2)

    sqk_b, sv_b, pos_b, x_b = jax.lax.optimization_barrier(
        (sqk, sv, sidx.astype(f32), x))
    o1 = _attn1(sqk_b.reshape(B * HEADS, S, DH), sv_b.reshape(B * HEADS, S, DH),
                pos_b.reshape(B * HEADS, S))
    o1 = o1.reshape(B, HEADS, S, DH).transpose(0, 2, 1, 3).reshape(B * S, DIM)
    logits = _tail(x_b.reshape(B * S, D), o1, lp, params).reshape(B, S, 128)
    return (logits[..., 0], logits[..., 1])
